# baseline (speedup 1.0000x reference)
import functools

import jax
import jax.numpy as jnp
from jax import lax
from jax.experimental import pallas as pl
from jax.experimental.pallas import tpu as pltpu
from jax.experimental.pallas import tpu_sc as plsc

VOCAB = 128100
HIDDEN = 768
MAX_POS = 8192
NUM_TOKENS = 16384
EPS = 1e-7

NC = 2
NS = 16
NW = NC * NS
L = 16
C = 16
NSL = HIDDEN // L

BT = 4096


_mesh = plsc.VectorSubcoreMesh(core_axis_name="c", subcore_axis_name="s")


def _make_gather_add(num_tokens):
    tok_per_w = num_tokens // NW
    nchunk = tok_per_w // C
    nstep = nchunk // 2

    @functools.partial(
        pl.kernel,
        mesh=_mesh,
        out_type=jax.ShapeDtypeStruct((num_tokens, HIDDEN), jnp.float32),
        compiler_params=pltpu.CompilerParams(needs_layout_passes=False),
        scratch_types=[
            pltpu.VMEM((tok_per_w,), jnp.int32),
            pltpu.VMEM((tok_per_w,), jnp.int32),
            pltpu.VMEM((C, HIDDEN), jnp.float32),
            pltpu.VMEM((C, HIDDEN), jnp.float32),
            pltpu.VMEM((C, HIDDEN), jnp.float32),
            pltpu.VMEM((C, HIDDEN), jnp.float32),
            pltpu.VMEM((C, HIDDEN), jnp.float32),
            pltpu.VMEM((C, HIDDEN), jnp.float32),
            pltpu.SemaphoreType.DMA,
            pltpu.SemaphoreType.DMA,
            pltpu.SemaphoreType.DMA,
            pltpu.SemaphoreType.DMA,
            pltpu.SemaphoreType.DMA,
        ],
    )
    def gather_add(ids_hbm, pids_hbm, wtab_hbm, ptab_hbm,
                   out_hbm, widx, pidx, wbuf0, wbuf1, pbuf0, pbuf1,
                   obuf0, obuf1, isem, gsem0, gsem1, osem0, osem1):
        wid = lax.axis_index("s") * NC + lax.axis_index("c")
        base = wid * tok_per_w

        pltpu.async_copy(ids_hbm.at[pl.ds(base, tok_per_w)], widx, isem)
        pltpu.async_copy(pids_hbm.at[pl.ds(base, tok_per_w)], pidx, isem)
        pltpu.make_async_copy(
            ids_hbm.at[pl.ds(base, tok_per_w)], widx, isem).wait()
        pltpu.make_async_copy(
            pids_hbm.at[pl.ds(base, tok_per_w)], pidx, isem).wait()

        bufs = ((wbuf0, pbuf0, obuf0, gsem0, osem0),
                (wbuf1, pbuf1, obuf1, gsem1, osem1))

        def issue_gathers(c, wbuf, pbuf, gsem):
            pltpu.async_copy(wtab_hbm.at[widx.at[pl.ds(c * C, C)]], wbuf, gsem)
            pltpu.async_copy(ptab_hbm.at[pidx.at[pl.ds(c * C, C)]], pbuf, gsem)

        issue_gathers(0, wbuf0, pbuf0, gsem0)
        issue_gathers(1, wbuf1, pbuf1, gsem1)

        def step(s, carry):
            for b in range(2):
                wbuf, pbuf, obuf, gsem, osem = bufs[b]
                c = 2 * s + b

                pltpu.make_async_copy(
                    wtab_hbm.at[widx.at[pl.ds(c * C, C)]], wbuf, gsem).wait()
                pltpu.make_async_copy(
                    ptab_hbm.at[pidx.at[pl.ds(c * C, C)]], pbuf, gsem).wait()

                @pl.when(s >= 1)
                def _wait_prev():
                    pltpu.make_async_copy(
                        obuf, out_hbm.at[pl.ds(base + (c - 2) * C, C)],
                        osem).wait()

                @plsc.parallel_loop(0, C, step=1, unroll=4)
                def _add(t):
                    for j in range(NSL):
                        obuf[t, pl.ds(j * L, L)] = (
                            wbuf[t, pl.ds(j * L, L)]
                            + pbuf[t, pl.ds(j * L, L)])

                @pl.when(s < nstep - 1)
                def _prefetch():
                    issue_gathers(c + 2, wbuf, pbuf, gsem)

                pltpu.async_copy(
                    obuf, out_hbm.at[pl.ds(base + c * C, C)], osem)
            return carry

        lax.fori_loop(0, nstep, step, 0)

        pltpu.make_async_copy(
            obuf0, out_hbm.at[pl.ds(base + (nchunk - 2) * C, C)],
            osem0).wait()
        pltpu.make_async_copy(
            obuf1, out_hbm.at[pl.ds(base + (nchunk - 1) * C, C)],
            osem1).wait()

    return gather_add


_gather_add_full = _make_gather_add(NUM_TOKENS)


def _ln_body(x_ref, g_ref, b_ref, o_ref):
    x = x_ref[...]
    mean = jnp.mean(x, axis=-1, keepdims=True)
    xc = x - mean
    var = jnp.mean(xc * xc, axis=-1, keepdims=True)
    inv = lax.rsqrt(var + jnp.float32(EPS))
    o_ref[...] = (xc * inv) * g_ref[...][None, :] + b_ref[...][None, :]


_ln_tc = pl.pallas_call(
    _ln_body,
    grid=(NUM_TOKENS // BT,),
    in_specs=[
        pl.BlockSpec((BT, HIDDEN), lambda i: (i, 0)),
        pl.BlockSpec((HIDDEN,), lambda i: (0,)),
        pl.BlockSpec((HIDDEN,), lambda i: (0,)),
    ],
    out_specs=pl.BlockSpec((BT, HIDDEN), lambda i: (i, 0)),
    out_shape=jax.ShapeDtypeStruct((NUM_TOKENS, HIDDEN), jnp.float32),
)


def kernel(input_ids, seq_lens, position_ids, word_embeddings,
           position_embeddings, ln_gamma, ln_beta):
    del seq_lens
    summed = _gather_add_full(input_ids, position_ids, word_embeddings,
                              position_embeddings)
    return _ln_tc(summed, ln_gamma, ln_beta)


# speedup vs baseline: 1.0756x; 1.0756x over previous
import functools

import jax
import jax.numpy as jnp
from jax import lax
from jax.experimental import pallas as pl
from jax.experimental.pallas import tpu as pltpu
from jax.experimental.pallas import tpu_sc as plsc

VOCAB = 128100
HIDDEN = 768
MAX_POS = 8192
NUM_TOKENS = 16384
EPS = 1e-7

NC = 2
NS = 16
NW = NC * NS
L = 16
C = 16
NSL = HIDDEN // L

BT = 4096


_mesh = plsc.VectorSubcoreMesh(core_axis_name="c", subcore_axis_name="s")


def _make_gather_add(num_tokens):
    tok_per_w = num_tokens // NW
    nchunk = tok_per_w // C
    nstep = nchunk // 2

    @functools.partial(
        pl.kernel,
        mesh=_mesh,
        out_type=jax.ShapeDtypeStruct((num_tokens, HIDDEN), jnp.float32),
        compiler_params=pltpu.CompilerParams(needs_layout_passes=False),
        scratch_types=[
            pltpu.VMEM((tok_per_w,), jnp.int32),
            pltpu.VMEM((tok_per_w,), jnp.int32),
            pltpu.VMEM((C, HIDDEN), jnp.float32),
            pltpu.VMEM((C, HIDDEN), jnp.float32),
            pltpu.VMEM((C, HIDDEN), jnp.float32),
            pltpu.VMEM((C, HIDDEN), jnp.float32),
            pltpu.VMEM((C, HIDDEN), jnp.float32),
            pltpu.VMEM((C, HIDDEN), jnp.float32),
            pltpu.SemaphoreType.DMA,
            pltpu.SemaphoreType.DMA,
            pltpu.SemaphoreType.DMA,
            pltpu.SemaphoreType.DMA,
            pltpu.SemaphoreType.DMA,
        ],
    )
    def gather_add(ids_hbm, pids_hbm, wtab_hbm, ptab_hbm,
                   out_hbm, widx, pidx, wbuf0, wbuf1, pbuf0, pbuf1,
                   obuf0, obuf1, isem, gsem0, gsem1, osem0, osem1):
        wid = lax.axis_index("s") * NC + lax.axis_index("c")
        base = wid * tok_per_w

        pltpu.async_copy(ids_hbm.at[pl.ds(base, tok_per_w)], widx, isem)
        pltpu.async_copy(pids_hbm.at[pl.ds(base, tok_per_w)], pidx, isem)
        pltpu.make_async_copy(
            ids_hbm.at[pl.ds(base, tok_per_w)], widx, isem).wait()
        pltpu.make_async_copy(
            pids_hbm.at[pl.ds(base, tok_per_w)], pidx, isem).wait()

        bufs = ((wbuf0, pbuf0, obuf0, gsem0, osem0),
                (wbuf1, pbuf1, obuf1, gsem1, osem1))

        def issue_gathers(c, wbuf, pbuf, gsem):
            pltpu.async_copy(wtab_hbm.at[widx.at[pl.ds(c * C, C)]], wbuf, gsem)
            pltpu.async_copy(ptab_hbm.at[pidx.at[pl.ds(c * C, C)]], pbuf, gsem)

        issue_gathers(0, wbuf0, pbuf0, gsem0)
        issue_gathers(1, wbuf1, pbuf1, gsem1)

        def step(s, carry):
            for b in range(2):
                wbuf, pbuf, obuf, gsem, osem = bufs[b]
                c = 2 * s + b

                pltpu.make_async_copy(
                    wtab_hbm.at[widx.at[pl.ds(c * C, C)]], wbuf, gsem).wait()
                pltpu.make_async_copy(
                    ptab_hbm.at[pidx.at[pl.ds(c * C, C)]], pbuf, gsem).wait()

                @pl.when(s >= 1)
                def _wait_prev():
                    pltpu.make_async_copy(
                        obuf, out_hbm.at[pl.ds(base + (c - 2) * C, C)],
                        osem).wait()

                @plsc.parallel_loop(0, C, step=1, unroll=1)
                def _add(t):
                    for j in range(NSL):
                        obuf[t, pl.ds(j * L, L)] = (
                            wbuf[t, pl.ds(j * L, L)]
                            + pbuf[t, pl.ds(j * L, L)])

                @pl.when(s < nstep - 1)
                def _prefetch():
                    issue_gathers(c + 2, wbuf, pbuf, gsem)

                pltpu.async_copy(
                    obuf, out_hbm.at[pl.ds(base + c * C, C)], osem)
            return carry

        lax.fori_loop(0, nstep, step, 0)

        pltpu.make_async_copy(
            obuf0, out_hbm.at[pl.ds(base + (nchunk - 2) * C, C)],
            osem0).wait()
        pltpu.make_async_copy(
            obuf1, out_hbm.at[pl.ds(base + (nchunk - 1) * C, C)],
            osem1).wait()

    return gather_add


_gather_add_full = _make_gather_add(NUM_TOKENS)


def _ln_body(x_ref, g_ref, b_ref, o_ref):
    x = x_ref[...]
    mean = jnp.mean(x, axis=-1, keepdims=True)
    xc = x - mean
    var = jnp.mean(xc * xc, axis=-1, keepdims=True)
    inv = lax.rsqrt(var + jnp.float32(EPS))
    o_ref[...] = (xc * inv) * g_ref[...][None, :] + b_ref[...][None, :]


_ln_tc = pl.pallas_call(
    _ln_body,
    grid=(NUM_TOKENS // BT,),
    in_specs=[
        pl.BlockSpec((BT, HIDDEN), lambda i: (i, 0)),
        pl.BlockSpec((HIDDEN,), lambda i: (0,)),
        pl.BlockSpec((HIDDEN,), lambda i: (0,)),
    ],
    out_specs=pl.BlockSpec((BT, HIDDEN), lambda i: (i, 0)),
    out_shape=jax.ShapeDtypeStruct((NUM_TOKENS, HIDDEN), jnp.float32),
)


def kernel(input_ids, seq_lens, position_ids, word_embeddings,
           position_embeddings, ln_gamma, ln_beta):
    del seq_lens
    summed = _gather_add_full(input_ids, position_ids, word_embeddings,
                              position_embeddings)
    return _ln_tc(summed, ln_gamma, ln_beta)


# speedup vs baseline: 1.0821x; 1.0061x over previous
import functools

import jax
import jax.numpy as jnp
from jax import lax
from jax.experimental import pallas as pl
from jax.experimental.pallas import tpu as pltpu
from jax.experimental.pallas import tpu_sc as plsc

VOCAB = 128100
HIDDEN = 768
MAX_POS = 8192
NUM_TOKENS = 16384
EPS = 1e-7

NC = 2
NS = 16
NW = NC * NS
L = 16
C = 16
NSL = HIDDEN // L

BT = 4096


_mesh = plsc.VectorSubcoreMesh(core_axis_name="c", subcore_axis_name="s")


def _make_gather_add(num_tokens):
    tok_per_w = num_tokens // NW
    nchunk = tok_per_w // C
    nstep = nchunk // 2

    @functools.partial(
        pl.kernel,
        mesh=_mesh,
        out_type=jax.ShapeDtypeStruct((num_tokens, HIDDEN), jnp.float32),
        compiler_params=pltpu.CompilerParams(needs_layout_passes=False),
        scratch_types=[
            pltpu.VMEM((tok_per_w,), jnp.int32),
            pltpu.VMEM((tok_per_w,), jnp.int32),
            pltpu.VMEM((C, HIDDEN), jnp.float32),
            pltpu.VMEM((C, HIDDEN), jnp.float32),
            pltpu.VMEM((C, HIDDEN), jnp.float32),
            pltpu.VMEM((C, HIDDEN), jnp.float32),
            pltpu.VMEM((C, HIDDEN), jnp.float32),
            pltpu.VMEM((C, HIDDEN), jnp.float32),
            pltpu.SemaphoreType.DMA,
            pltpu.SemaphoreType.DMA,
            pltpu.SemaphoreType.DMA,
            pltpu.SemaphoreType.DMA,
            pltpu.SemaphoreType.DMA,
        ],
    )
    def gather_add(ids_hbm, pids_hbm, wtab_hbm, ptab_hbm,
                   out_hbm, widx, pidx, wbuf0, wbuf1, pbuf0, pbuf1,
                   obuf0, obuf1, isem, gsem0, gsem1, osem0, osem1):
        wid = lax.axis_index("s") * NC + lax.axis_index("c")
        base = wid * tok_per_w

        pltpu.async_copy(ids_hbm.at[pl.ds(base, tok_per_w)], widx, isem)
        pltpu.async_copy(pids_hbm.at[pl.ds(base, tok_per_w)], pidx, isem)
        pltpu.make_async_copy(
            ids_hbm.at[pl.ds(base, tok_per_w)], widx, isem).wait()
        pltpu.make_async_copy(
            pids_hbm.at[pl.ds(base, tok_per_w)], pidx, isem).wait()

        bufs = ((wbuf0, pbuf0, obuf0, gsem0, osem0),
                (wbuf1, pbuf1, obuf1, gsem1, osem1))

        def issue_gathers(c, wbuf, pbuf, gsem):
            pltpu.async_copy(wtab_hbm.at[widx.at[pl.ds(c * C, C)]], wbuf, gsem)
            pltpu.async_copy(ptab_hbm.at[pidx.at[pl.ds(c * C, C)]], pbuf, gsem)

        issue_gathers(0, wbuf0, pbuf0, gsem0)
        issue_gathers(1, wbuf1, pbuf1, gsem1)

        def step(s, carry):
            for b in range(2):
                wbuf, pbuf, obuf, gsem, osem = bufs[b]
                c = 2 * s + b

                pltpu.make_async_copy(
                    wtab_hbm.at[widx.at[pl.ds(c * C, C)]], wbuf, gsem).wait()
                pltpu.make_async_copy(
                    ptab_hbm.at[pidx.at[pl.ds(c * C, C)]], pbuf, gsem).wait()

                @pl.when(s >= 1)
                def _wait_prev():
                    pltpu.make_async_copy(
                        obuf, out_hbm.at[pl.ds(base + (c - 2) * C, C)],
                        osem).wait()

                @plsc.parallel_loop(0, C, step=1, unroll=2)
                def _add(t):
                    for j in range(NSL):
                        obuf[t, pl.ds(j * L, L)] = (
                            wbuf[t, pl.ds(j * L, L)]
                            + pbuf[t, pl.ds(j * L, L)])

                @pl.when(s < nstep - 1)
                def _prefetch():
                    issue_gathers(c + 2, wbuf, pbuf, gsem)

                pltpu.async_copy(
                    obuf, out_hbm.at[pl.ds(base + c * C, C)], osem)
            return carry

        lax.fori_loop(0, nstep, step, 0)

        pltpu.make_async_copy(
            obuf0, out_hbm.at[pl.ds(base + (nchunk - 2) * C, C)],
            osem0).wait()
        pltpu.make_async_copy(
            obuf1, out_hbm.at[pl.ds(base + (nchunk - 1) * C, C)],
            osem1).wait()

    return gather_add


_gather_add_full = _make_gather_add(NUM_TOKENS)


def _ln_body(x_ref, g_ref, b_ref, o_ref):
    x = x_ref[...]
    mean = jnp.mean(x, axis=-1, keepdims=True)
    xc = x - mean
    var = jnp.mean(xc * xc, axis=-1, keepdims=True)
    inv = lax.rsqrt(var + jnp.float32(EPS))
    o_ref[...] = (xc * inv) * g_ref[...][None, :] + b_ref[...][None, :]


_ln_tc = pl.pallas_call(
    _ln_body,
    grid=(NUM_TOKENS // BT,),
    in_specs=[
        pl.BlockSpec((BT, HIDDEN), lambda i: (i, 0)),
        pl.BlockSpec((HIDDEN,), lambda i: (0,)),
        pl.BlockSpec((HIDDEN,), lambda i: (0,)),
    ],
    out_specs=pl.BlockSpec((BT, HIDDEN), lambda i: (i, 0)),
    out_shape=jax.ShapeDtypeStruct((NUM_TOKENS, HIDDEN), jnp.float32),
)


def kernel(input_ids, seq_lens, position_ids, word_embeddings,
           position_embeddings, ln_gamma, ln_beta):
    del seq_lens
    summed = _gather_add_full(input_ids, position_ids, word_embeddings,
                              position_embeddings)
    return _ln_tc(summed, ln_gamma, ln_beta)


# speedup vs baseline: 1.0837x; 1.0015x over previous
import functools

import jax
import jax.numpy as jnp
from jax import lax
from jax.experimental import pallas as pl
from jax.experimental.pallas import tpu as pltpu
from jax.experimental.pallas import tpu_sc as plsc

VOCAB = 128100
HIDDEN = 768
MAX_POS = 8192
NUM_TOKENS = 16384
EPS = 1e-7

NC = 2
NS = 16
NW = NC * NS
L = 16
C = 16
NSL = HIDDEN // L

BT = 4096


_mesh = plsc.VectorSubcoreMesh(core_axis_name="c", subcore_axis_name="s")


def _make_gather_add(num_tokens):
    tok_per_w = num_tokens // NW
    nchunk = tok_per_w // C
    nstep = nchunk // 2

    @functools.partial(
        pl.kernel,
        mesh=_mesh,
        out_type=jax.ShapeDtypeStruct((num_tokens, HIDDEN), jnp.float32),
        scratch_types=[
            pltpu.VMEM((tok_per_w,), jnp.int32),
            pltpu.VMEM((tok_per_w,), jnp.int32),
            pltpu.VMEM((C, HIDDEN), jnp.float32),
            pltpu.VMEM((C, HIDDEN), jnp.float32),
            pltpu.VMEM((C, HIDDEN), jnp.float32),
            pltpu.VMEM((C, HIDDEN), jnp.float32),
            pltpu.VMEM((C, HIDDEN), jnp.float32),
            pltpu.VMEM((C, HIDDEN), jnp.float32),
            pltpu.SemaphoreType.DMA,
            pltpu.SemaphoreType.DMA,
            pltpu.SemaphoreType.DMA,
            pltpu.SemaphoreType.DMA,
            pltpu.SemaphoreType.DMA,
        ],
    )
    def gather_add(ids_hbm, pids_hbm, wtab_hbm, ptab_hbm,
                   out_hbm, widx, pidx, wbuf0, wbuf1, pbuf0, pbuf1,
                   obuf0, obuf1, isem, gsem0, gsem1, osem0, osem1):
        wid = lax.axis_index("s") * NC + lax.axis_index("c")
        base = wid * tok_per_w

        pltpu.async_copy(ids_hbm.at[pl.ds(base, tok_per_w)], widx, isem)
        pltpu.async_copy(pids_hbm.at[pl.ds(base, tok_per_w)], pidx, isem)
        pltpu.make_async_copy(
            ids_hbm.at[pl.ds(base, tok_per_w)], widx, isem).wait()
        pltpu.make_async_copy(
            pids_hbm.at[pl.ds(base, tok_per_w)], pidx, isem).wait()

        bufs = ((wbuf0, pbuf0, obuf0, gsem0, osem0),
                (wbuf1, pbuf1, obuf1, gsem1, osem1))

        def issue_gathers(c, wbuf, pbuf, gsem):
            pltpu.async_copy(wtab_hbm.at[widx.at[pl.ds(c * C, C)]], wbuf, gsem)
            pltpu.async_copy(ptab_hbm.at[pidx.at[pl.ds(c * C, C)]], pbuf, gsem)

        issue_gathers(0, wbuf0, pbuf0, gsem0)
        issue_gathers(1, wbuf1, pbuf1, gsem1)

        def step(s, carry):
            for b in range(2):
                wbuf, pbuf, obuf, gsem, osem = bufs[b]
                c = 2 * s + b

                pltpu.make_async_copy(
                    wtab_hbm.at[widx.at[pl.ds(c * C, C)]], wbuf, gsem).wait()
                pltpu.make_async_copy(
                    ptab_hbm.at[pidx.at[pl.ds(c * C, C)]], pbuf, gsem).wait()

                @pl.when(s >= 1)
                def _wait_prev():
                    pltpu.make_async_copy(
                        obuf, out_hbm.at[pl.ds(base + (c - 2) * C, C)],
                        osem).wait()

                @plsc.parallel_loop(0, C, step=1, unroll=2)
                def _add(t):
                    for j in range(NSL):
                        obuf[t, pl.ds(j * L, L)] = (
                            wbuf[t, pl.ds(j * L, L)]
                            + pbuf[t, pl.ds(j * L, L)])

                @pl.when(s < nstep - 1)
                def _prefetch():
                    issue_gathers(c + 2, wbuf, pbuf, gsem)

                pltpu.async_copy(
                    obuf, out_hbm.at[pl.ds(base + c * C, C)], osem)
            return carry

        lax.fori_loop(0, nstep, step, 0)

        pltpu.make_async_copy(
            obuf0, out_hbm.at[pl.ds(base + (nchunk - 2) * C, C)],
            osem0).wait()
        pltpu.make_async_copy(
            obuf1, out_hbm.at[pl.ds(base + (nchunk - 1) * C, C)],
            osem1).wait()

    return gather_add


_gather_add_full = _make_gather_add(NUM_TOKENS)


def _ln_body(x_ref, g_ref, b_ref, o_ref):
    x = x_ref[...]
    mean = jnp.mean(x, axis=-1, keepdims=True)
    xc = x - mean
    var = jnp.mean(xc * xc, axis=-1, keepdims=True)
    inv = lax.rsqrt(var + jnp.float32(EPS))
    o_ref[...] = (xc * inv) * g_ref[...][None, :] + b_ref[...][None, :]


_ln_tc = pl.pallas_call(
    _ln_body,
    grid=(NUM_TOKENS // BT,),
    in_specs=[
        pl.BlockSpec((BT, HIDDEN), lambda i: (i, 0)),
        pl.BlockSpec((HIDDEN,), lambda i: (0,)),
        pl.BlockSpec((HIDDEN,), lambda i: (0,)),
    ],
    out_specs=pl.BlockSpec((BT, HIDDEN), lambda i: (i, 0)),
    out_shape=jax.ShapeDtypeStruct((NUM_TOKENS, HIDDEN), jnp.float32),
)


def kernel(input_ids, seq_lens, position_ids, word_embeddings,
           position_embeddings, ln_gamma, ln_beta):
    del seq_lens
    summed = _gather_add_full(input_ids, position_ids, word_embeddings,
                              position_embeddings)
    return _ln_tc(summed, ln_gamma, ln_beta)
